# baseline (speedup 1.0000x reference)
import functools

import jax
import jax.numpy as jnp
from jax import lax
from jax.experimental import pallas as pl
from jax.experimental.pallas import tpu as pltpu
from jax.experimental.pallas import tpu_sc as plsc

_CHUNK = 96
_NT = 16
_NC = 2


def _round_up(v, m):
    return (v + m - 1) // m * m


@functools.partial(jax.jit, static_argnames=("n", "e_pad"))
def _sc_degree(col2d, ew2d, zeros_n, *, n, e_pad):
    nw = _NC * _NT
    nct = e_pad // _CHUNK // nw
    blk = _round_up(-(-n // _NT), 128)
    n_pad = blk * _NT
    ring = 8
    mesh = plsc.VectorSubcoreMesh(core_axis_name="c", subcore_axis_name="s")

    @functools.partial(
        pl.kernel,
        mesh=mesh,
        out_type=jax.ShapeDtypeStruct((_NC, 1, n_pad), jnp.float32),
        scratch_types=[
            pltpu.VMEM((nct, 1, _CHUNK), jnp.int32),
            pltpu.VMEM((nct, 1, _CHUNK), jnp.float32),
            pltpu.VMEM_SHARED((n_pad,), jnp.float32),
            pltpu.SemaphoreType.DMA,
        ],
    )
    def deg_kernel(col_hbm, ew_hbm, z_hbm, out_hbm, cidx_all, ew_all, acc, sem):
        cid = lax.axis_index("c")
        sid = lax.axis_index("s")
        wid = sid * _NC + cid

        pltpu.sync_copy(z_hbm.at[pl.ds(sid * blk, blk)],
                        acc.at[pl.ds(sid * blk, blk)])
        pltpu.sync_copy(col_hbm.at[pl.ds(wid * nct, nct)], cidx_all)
        pltpu.sync_copy(ew_hbm.at[pl.ds(wid * nct, nct)], ew_all)
        plsc.subcore_barrier()

        def chunk_body(j, carry):
            pltpu.async_copy(ew_all.at[j, 0], acc.at[cidx_all.at[j, 0]], sem,
                             add=True)

            @pl.when(j >= ring)
            def _():
                pltpu.make_async_copy(z_hbm.at[pl.ds(0, _CHUNK)],
                                      ew_all.at[0, 0], sem).wait()

            return carry

        lax.fori_loop(0, nct, chunk_body, 0)
        for _ in range(min(ring, nct)):
            pltpu.make_async_copy(z_hbm.at[pl.ds(0, _CHUNK)],
                                  ew_all.at[0, 0], sem).wait()
        plsc.subcore_barrier()

        pltpu.sync_copy(acc.at[pl.ds(sid * blk, blk)],
                        out_hbm.at[cid, 0, pl.ds(sid * blk, blk)])

    return deg_kernel(col2d, ew2d, zeros_n)


@functools.partial(jax.jit, static_argnames=("n", "e_pad", "fh_active"))
def _sc_spmm(g_tab, packed2d, ew2d, zeros_nf, *, n, e_pad, fh_active):
    fh = g_tab.shape[1]
    nw = _NC * _NT
    nct = e_pad // _CHUNK // nw
    blk = _round_up(-(-n // _NT), 128)
    n_pad = blk * _NT
    mesh = plsc.VectorSubcoreMesh(core_axis_name="c", subcore_axis_name="s")

    @functools.partial(
        pl.kernel,
        mesh=mesh,
        out_type=jax.ShapeDtypeStruct((_NC, n_pad, fh), jnp.float32),
        compiler_params=(None if fh >= 128 else
                         pltpu.CompilerParams(use_tc_tiling_on_sc=False)),
        scratch_types=[
            pltpu.VMEM((3, 1, _CHUNK), jnp.int32),
            pltpu.VMEM((3, 1, _CHUNK), jnp.float32),
            pltpu.VMEM((3, 1, _CHUNK), jnp.int32),
            pltpu.VMEM((3, 1, _CHUNK), jnp.int32),
            pltpu.VMEM((3, _CHUNK, fh), jnp.float32),
            pltpu.VMEM_SHARED((n_pad, fh), jnp.float32),
            pltpu.SemaphoreType.DMA,
            pltpu.SemaphoreType.DMA,
            pltpu.SemaphoreType.DMA,
            pltpu.SemaphoreType.DMA,
            pltpu.SemaphoreType.DMA,
            pltpu.SemaphoreType.DMA,
            pltpu.SemaphoreType.DMA,
            pltpu.SemaphoreType.DMA,
            pltpu.SemaphoreType.DMA,
        ],
    )
    def spmm_kernel(g_hbm, packed_hbm, ew_hbm, z_hbm, out_hbm,
                    pk_ring, ew_ring, ridx_ring, cidx_ring, rows3, acc,
                    gs0, gs1, gs2, ss0, ss1, ss2, is0, is1, is2):
        cid = lax.axis_index("c")
        sid = lax.axis_index("s")
        wid = sid * _NC + cid
        gsems = (gs0, gs1, gs2)
        ssems = (ss0, ss1, ss2)
        isems = (is0, is1, is2)
        cbase = wid * nct

        pltpu.sync_copy(z_hbm.at[pl.ds(sid * blk, blk)],
                        acc.at[pl.ds(sid * blk, blk)])

        def idxload_start(j, b):
            pltpu.async_copy(packed_hbm.at[pl.ds(cbase + j, 1)],
                             pk_ring.at[pl.ds(b, 1)], isems[b])
            pltpu.async_copy(ew_hbm.at[pl.ds(cbase + j, 1)],
                             ew_ring.at[pl.ds(b, 1)], isems[b])

        def idxload_wait(b):
            pltpu.make_async_copy(packed_hbm.at[pl.ds(0, 1)],
                                  pk_ring.at[pl.ds(b, 1)], isems[b]).wait()
            pltpu.make_async_copy(ew_hbm.at[pl.ds(0, 1)],
                                  ew_ring.at[pl.ds(b, 1)], isems[b]).wait()

        def unpack(b):
            for q in range(_CHUNK // 16):
                sl = pl.ds(q * 16, 16)
                p = pk_ring[b, 0, sl]
                ridx_ring[b, 0, sl] = p & 0x3FFF
                cidx_ring[b, 0, sl] = lax.shift_right_logical(p, 14)

        def gather_start(b):
            pltpu.async_copy(g_hbm.at[ridx_ring.at[b, 0]], rows3.at[b],
                             gsems[b])

        def gather_wait(b):
            pltpu.make_async_copy(g_hbm.at[pl.ds(0, _CHUNK)], rows3.at[b],
                                  gsems[b]).wait()

        def scatter_start(b):
            pltpu.async_copy(rows3.at[b], acc.at[cidx_ring.at[b, 0]],
                             ssems[b], add=True)

        def scatter_wait(b):
            pltpu.make_async_copy(g_hbm.at[pl.ds(0, _CHUNK)], rows3.at[b],
                                  ssems[b]).wait()

        def compute(b):
            def group_body(gi, c2):
                ew16 = ew_ring[b, 0, pl.ds(gi * 16, 16)]
                for i in range(16):
                    s = ew16[i]
                    e = gi * 16 + i
                    for jj in range(fh_active // 16):
                        fs = pl.ds(jj * 16, 16)
                        rows3[b, e, fs] = rows3[b, e, fs] * s
                return c2

            lax.fori_loop(0, _CHUNK // 16, group_body, 0)

        idxload_start(0, 0)
        idxload_start(1, 1)
        idxload_start(2, 2)
        idxload_wait(0)
        unpack(0)
        gather_start(0)
        idxload_wait(1)
        unpack(1)
        gather_start(1)
        plsc.subcore_barrier()

        ntrip = (nct - 1) // 3

        def triple(t, carry):
            for k in range(3):
                i = 3 * t + k
                b = k
                b2 = (k + 2) % 3
                gather_wait(b)
                compute(b)
                scatter_start(b)
                if k == 0:
                    @pl.when(t > 0)
                    def _():
                        scatter_wait(b2)
                else:
                    scatter_wait(b2)

                @pl.when(i + 3 < nct)
                def _():
                    idxload_start(i + 3, b)

                @pl.when(i + 2 < nct)
                def _():
                    idxload_wait(b2)
                    unpack(b2)
                    gather_start(b2)
            return carry

        lax.fori_loop(0, ntrip, triple, 0)

        bt = (nct - 1) % 3
        gather_wait(bt)
        compute(bt)
        scatter_start(bt)
        scatter_wait((nct - 2) % 3)
        scatter_wait((nct - 1) % 3)

        plsc.subcore_barrier()
        pltpu.sync_copy(acc.at[pl.ds(sid * blk, blk)],
                        out_hbm.at[cid, pl.ds(sid * blk, blk)])

    return spmm_kernel(g_tab, packed2d, ew2d, zeros_nf)


def _tc_pre(x, W1, dp0, dp1):
    n, _ = x.shape
    h = W1.shape[1]

    def body(x_ref, w_ref, a_ref, b_ref, g_ref, dinv_ref):
        deg = a_ref[...] + b_ref[...] + 1.0
        dinv = lax.rsqrt(deg)
        t = jnp.dot(x_ref[...], w_ref[...], preferred_element_type=jnp.float32)
        g_ref[...] = dinv * t
        dinv_ref[...] = dinv

    return pl.pallas_call(
        body,
        out_shape=[jax.ShapeDtypeStruct((n, h), jnp.float32),
                   jax.ShapeDtypeStruct((n, 1), jnp.float32)],
    )(x, W1, dp0, dp1)


def _tc_mid(s1p, g1, dinv, b1r, W2):
    n = dinv.shape[0]
    h = g1.shape[1]
    c = W2.shape[1]

    def body(s_ref, g_ref, d_ref, b_ref, w_ref, out_ref):
        dinv = d_ref[...]
        h1 = jnp.maximum(dinv * (s_ref[0, :n] + s_ref[1, :n] + g_ref[...])
                         + b_ref[...], 0.0)
        t2 = jnp.dot(h1, w_ref[...], preferred_element_type=jnp.float32)
        out_ref[...] = dinv * t2

    return pl.pallas_call(
        body,
        out_shape=jax.ShapeDtypeStruct((n, c), jnp.float32),
    )(s1p, g1, dinv, b1r, W2)


def _tc_post(s2p, g2, dinv, b2r, c):
    n = dinv.shape[0]

    def body(s_ref, g_ref, d_ref, b_ref, out_ref):
        pre = s_ref[0, :n] + s_ref[1, :n] + g_ref[...]
        out_ref[...] = d_ref[...] * pre + b_ref[...]

    return pl.pallas_call(
        body,
        out_shape=jax.ShapeDtypeStruct((n, c), jnp.float32),
    )(s2p, g2, dinv, b2r)


def kernel(x, edge_index, edge_weight, W1, b1, W2, b2):
    n, _ = x.shape
    e = edge_index.shape[1]
    h = W1.shape[1]
    c = W2.shape[1]

    quant = _NC * _NT * _CHUNK
    e_pad = _round_up(e, quant)
    while (e_pad // quant) % 3 != 1:
        e_pad += quant
    pad = e_pad - e
    spread = jnp.arange(pad, dtype=jnp.int32) % n
    rowv = jnp.concatenate([edge_index[0], spread])
    colv = jnp.concatenate([edge_index[1], spread])
    nchunks = e_pad // _CHUNK
    packed = (rowv | (colv << 14)).reshape(nchunks, 1, _CHUNK)
    col = colv.reshape(nchunks, 1, _CHUNK)
    ew = jnp.concatenate(
        [edge_weight, jnp.zeros((pad,), jnp.float32)]).reshape(nchunks, 1, _CHUNK)

    n_pad = _round_up(-(-n // _NT), 128) * _NT
    zeros_nh = jnp.zeros((n_pad, h), jnp.float32)

    degp = _sc_degree(col, ew, zeros_nh[:, 0], n=n, e_pad=e_pad)
    dp0 = degp[0, 0, :n].reshape(n, 1)
    dp1 = degp[1, 0, :n].reshape(n, 1)

    g1, dinv = _tc_pre(x, W1, dp0, dp1)
    s1p = _sc_spmm(g1, packed, ew, zeros_nh, n=n, e_pad=e_pad, fh_active=h)
    g2 = _tc_mid(s1p, g1, dinv, b1.reshape(1, h), W2)
    s2p = _sc_spmm(g2, packed, ew, zeros_nh[:, : c], n=n, e_pad=e_pad,
                   fh_active=c)
    return _tc_post(s2p, g2, dinv, b2.reshape(1, c), c)


# speedup vs baseline: 1.0123x; 1.0123x over previous
import functools

import jax
import jax.numpy as jnp
from jax import lax
from jax.experimental import pallas as pl
from jax.experimental.pallas import tpu as pltpu
from jax.experimental.pallas import tpu_sc as plsc

_CHUNK = 96
_NT = 16
_NC = 2


def _round_up(v, m):
    return (v + m - 1) // m * m


@functools.partial(jax.jit, static_argnames=("n", "e_pad"))
def _sc_degree(col2d, ew2d, zeros_n, *, n, e_pad):
    nw = _NC * _NT
    nct = e_pad // _CHUNK // nw
    blk = _round_up(-(-n // _NT), 128)
    n_pad = blk * _NT
    ring = 8
    mesh = plsc.VectorSubcoreMesh(core_axis_name="c", subcore_axis_name="s")

    @functools.partial(
        pl.kernel,
        mesh=mesh,
        out_type=jax.ShapeDtypeStruct((_NC, 1, n_pad), jnp.float32),
        scratch_types=[
            pltpu.VMEM((nct, 1, _CHUNK), jnp.int32),
            pltpu.VMEM((nct, 1, _CHUNK), jnp.float32),
            pltpu.VMEM_SHARED((n_pad,), jnp.float32),
            pltpu.SemaphoreType.DMA,
        ],
    )
    def deg_kernel(col_hbm, ew_hbm, z_hbm, out_hbm, cidx_all, ew_all, acc, sem):
        cid = lax.axis_index("c")
        sid = lax.axis_index("s")
        wid = sid * _NC + cid

        pltpu.sync_copy(z_hbm.at[pl.ds(sid * blk, blk)],
                        acc.at[pl.ds(sid * blk, blk)])
        pltpu.sync_copy(col_hbm.at[pl.ds(wid * nct, nct)], cidx_all)
        pltpu.sync_copy(ew_hbm.at[pl.ds(wid * nct, nct)], ew_all)
        plsc.subcore_barrier()

        def chunk_body(j, carry):
            pltpu.async_copy(ew_all.at[j, 0], acc.at[cidx_all.at[j, 0]], sem,
                             add=True)

            @pl.when(j >= ring)
            def _():
                pltpu.make_async_copy(z_hbm.at[pl.ds(0, _CHUNK)],
                                      ew_all.at[0, 0], sem).wait()

            return carry

        lax.fori_loop(0, nct, chunk_body, 0)
        for _ in range(min(ring, nct)):
            pltpu.make_async_copy(z_hbm.at[pl.ds(0, _CHUNK)],
                                  ew_all.at[0, 0], sem).wait()
        plsc.subcore_barrier()

        pltpu.sync_copy(acc.at[pl.ds(sid * blk, blk)],
                        out_hbm.at[cid, 0, pl.ds(sid * blk, blk)])

    return deg_kernel(col2d, ew2d, zeros_n)


@functools.partial(jax.jit, static_argnames=("n", "e_pad", "fh_active"))
def _sc_spmm(g_tab, packed2d, ew2d, zeros_nf, *, n, e_pad, fh_active):
    fh = g_tab.shape[1]
    nw = _NC * _NT
    nct = e_pad // _CHUNK // nw
    blk = _round_up(-(-n // _NT), 128)
    n_pad = blk * _NT
    mesh = plsc.VectorSubcoreMesh(core_axis_name="c", subcore_axis_name="s")

    @functools.partial(
        pl.kernel,
        mesh=mesh,
        out_type=jax.ShapeDtypeStruct((_NC, n_pad, fh), jnp.float32),
        scratch_types=[
            pltpu.VMEM((3, 1, _CHUNK), jnp.int32),
            pltpu.VMEM((3, 1, _CHUNK), jnp.float32),
            pltpu.VMEM((3, 1, _CHUNK), jnp.int32),
            pltpu.VMEM((3, 1, _CHUNK), jnp.int32),
            pltpu.VMEM((3, _CHUNK, fh), jnp.float32),
            pltpu.VMEM_SHARED((n_pad, fh), jnp.float32),
            pltpu.SemaphoreType.DMA,
            pltpu.SemaphoreType.DMA,
            pltpu.SemaphoreType.DMA,
            pltpu.SemaphoreType.DMA,
            pltpu.SemaphoreType.DMA,
            pltpu.SemaphoreType.DMA,
            pltpu.SemaphoreType.DMA,
            pltpu.SemaphoreType.DMA,
            pltpu.SemaphoreType.DMA,
        ],
    )
    def spmm_kernel(g_hbm, packed_hbm, ew_hbm, z_hbm, out_hbm,
                    pk_ring, ew_ring, ridx_ring, cidx_ring, rows3, acc,
                    gs0, gs1, gs2, ss0, ss1, ss2, is0, is1, is2):
        cid = lax.axis_index("c")
        sid = lax.axis_index("s")
        wid = sid * _NC + cid
        gsems = (gs0, gs1, gs2)
        ssems = (ss0, ss1, ss2)
        isems = (is0, is1, is2)
        cbase = wid * nct

        pltpu.sync_copy(z_hbm.at[pl.ds(sid * blk, blk)],
                        acc.at[pl.ds(sid * blk, blk)])

        def idxload_start(j, b):
            pltpu.async_copy(packed_hbm.at[pl.ds(cbase + j, 1)],
                             pk_ring.at[pl.ds(b, 1)], isems[b])
            pltpu.async_copy(ew_hbm.at[pl.ds(cbase + j, 1)],
                             ew_ring.at[pl.ds(b, 1)], isems[b])

        def idxload_wait(b):
            pltpu.make_async_copy(packed_hbm.at[pl.ds(0, 1)],
                                  pk_ring.at[pl.ds(b, 1)], isems[b]).wait()
            pltpu.make_async_copy(ew_hbm.at[pl.ds(0, 1)],
                                  ew_ring.at[pl.ds(b, 1)], isems[b]).wait()

        def unpack(b):
            for q in range(_CHUNK // 16):
                sl = pl.ds(q * 16, 16)
                p = pk_ring[b, 0, sl]
                ridx_ring[b, 0, sl] = p & 0x3FFF
                cidx_ring[b, 0, sl] = lax.shift_right_logical(p, 14)

        def gather_start(b):
            pltpu.async_copy(g_hbm.at[ridx_ring.at[b, 0]], rows3.at[b],
                             gsems[b])

        def gather_wait(b):
            pltpu.make_async_copy(g_hbm.at[pl.ds(0, _CHUNK)], rows3.at[b],
                                  gsems[b]).wait()

        def scatter_start(b):
            pltpu.async_copy(rows3.at[b], acc.at[cidx_ring.at[b, 0]],
                             ssems[b], add=True)

        def scatter_wait(b):
            pltpu.make_async_copy(g_hbm.at[pl.ds(0, _CHUNK)], rows3.at[b],
                                  ssems[b]).wait()

        def compute(b):
            def group_body(gi, c2):
                ew16 = ew_ring[b, 0, pl.ds(gi * 16, 16)]
                for i in range(16):
                    s = ew16[i]
                    e = gi * 16 + i
                    for jj in range(fh_active // 16):
                        fs = pl.ds(jj * 16, 16)
                        rows3[b, e, fs] = rows3[b, e, fs] * s
                return c2

            lax.fori_loop(0, _CHUNK // 16, group_body, 0)

        idxload_start(0, 0)
        idxload_start(1, 1)
        idxload_start(2, 2)
        idxload_wait(0)
        unpack(0)
        gather_start(0)
        idxload_wait(1)
        unpack(1)
        gather_start(1)
        plsc.subcore_barrier()

        ntrip = (nct - 1) // 3

        def triple(t, carry):
            for k in range(3):
                i = 3 * t + k
                b = k
                b2 = (k + 2) % 3
                gather_wait(b)
                compute(b)
                scatter_start(b)
                if k == 0:
                    @pl.when(t > 0)
                    def _():
                        scatter_wait(b2)
                else:
                    scatter_wait(b2)

                @pl.when(i + 3 < nct)
                def _():
                    idxload_start(i + 3, b)

                @pl.when(i + 2 < nct)
                def _():
                    idxload_wait(b2)
                    unpack(b2)
                    gather_start(b2)
            return carry

        lax.fori_loop(0, ntrip, triple, 0)

        bt = (nct - 1) % 3
        gather_wait(bt)
        compute(bt)
        scatter_start(bt)
        scatter_wait((nct - 2) % 3)
        scatter_wait((nct - 1) % 3)

        plsc.subcore_barrier()
        pltpu.sync_copy(acc.at[pl.ds(sid * blk, blk)],
                        out_hbm.at[cid, pl.ds(sid * blk, blk)])

    return spmm_kernel(g_tab, packed2d, ew2d, zeros_nf)


def _tc_pre(x, W1, dp0, dp1):
    n, _ = x.shape
    h = W1.shape[1]

    def body(x_ref, w_ref, a_ref, b_ref, g_ref, dinv_ref):
        deg = a_ref[...] + b_ref[...] + 1.0
        dinv = lax.rsqrt(deg)
        t = jnp.dot(x_ref[...], w_ref[...], preferred_element_type=jnp.float32)
        g_ref[...] = dinv * t
        dinv_ref[...] = dinv

    return pl.pallas_call(
        body,
        out_shape=[jax.ShapeDtypeStruct((n, h), jnp.float32),
                   jax.ShapeDtypeStruct((n, 1), jnp.float32)],
    )(x, W1, dp0, dp1)


def _tc_mid(s1p, g1, dinv, b1r, W2):
    n = dinv.shape[0]
    h = g1.shape[1]
    c = W2.shape[1]

    def body(s_ref, g_ref, d_ref, b_ref, w_ref, out_ref):
        dinv = d_ref[...]
        h1 = jnp.maximum(dinv * (s_ref[0, :n] + s_ref[1, :n] + g_ref[...])
                         + b_ref[...], 0.0)
        t2 = jnp.dot(h1, w_ref[...], preferred_element_type=jnp.float32)
        out_ref[...] = jnp.concatenate(
            [dinv * t2, jnp.zeros((n, h - c), jnp.float32)], axis=1)

    return pl.pallas_call(
        body,
        out_shape=jax.ShapeDtypeStruct((n, h), jnp.float32),
    )(s1p, g1, dinv, b1r, W2)


def _tc_post(s2p, g2pad, dinv, b2r, c):
    n = dinv.shape[0]

    def body(s_ref, g_ref, d_ref, b_ref, out_ref):
        pre = s_ref[0, :n, :c] + s_ref[1, :n, :c] + g_ref[:, :c]
        out_ref[...] = d_ref[...] * pre + b_ref[...]

    return pl.pallas_call(
        body,
        out_shape=jax.ShapeDtypeStruct((n, c), jnp.float32),
    )(s2p, g2pad, dinv, b2r)


def kernel(x, edge_index, edge_weight, W1, b1, W2, b2):
    n, _ = x.shape
    e = edge_index.shape[1]
    h = W1.shape[1]
    c = W2.shape[1]

    quant = _NC * _NT * _CHUNK
    e_pad = _round_up(e, quant)
    while (e_pad // quant) % 3 != 1:
        e_pad += quant
    pad = e_pad - e
    spread = jnp.arange(pad, dtype=jnp.int32) % n
    rowv = jnp.concatenate([edge_index[0], spread])
    colv = jnp.concatenate([edge_index[1], spread])
    nchunks = e_pad // _CHUNK
    packed = (rowv | (colv << 14)).reshape(nchunks, 1, _CHUNK)
    col = colv.reshape(nchunks, 1, _CHUNK)
    ew = jnp.concatenate(
        [edge_weight, jnp.zeros((pad,), jnp.float32)]).reshape(nchunks, 1, _CHUNK)

    n_pad = _round_up(-(-n // _NT), 128) * _NT
    zeros_nh = jnp.zeros((n_pad, h), jnp.float32)

    degp = _sc_degree(col, ew, zeros_nh[:, 0], n=n, e_pad=e_pad)
    dp0 = degp[0, 0, :n].reshape(n, 1)
    dp1 = degp[1, 0, :n].reshape(n, 1)

    g1, dinv = _tc_pre(x, W1, dp0, dp1)
    s1p = _sc_spmm(g1, packed, ew, zeros_nh, n=n, e_pad=e_pad, fh_active=h)
    g2pad = _tc_mid(s1p, g1, dinv, b1.reshape(1, h), W2)
    s2p = _sc_spmm(g2pad, packed, ew, zeros_nh, n=n, e_pad=e_pad, fh_active=c)
    return _tc_post(s2p, g2pad, dinv, b2.reshape(1, c), c)
